# Initial kernel scaffold; baseline (speedup 1.0000x reference)
import jax
import jax.numpy as jnp
from jax.experimental import pallas as pl


def kernel(feature, beta, time, fs, hs, ts, father, W_ih, W_hh, b_ih, b_hh, W_o, b_o):
    raise NotImplementedError("write your pallas kernel here")


# speedup vs baseline: 1.1334x; 1.1334x over previous
import math

import jax
import jax.numpy as jnp
from jax.experimental import pallas as pl
from jax.experimental.pallas import tpu as pltpu

D_IN = 512
HID = 1024
D_OUT = 512
TOPK = 1024
NROWS = 20000
DECAY = 0.99

_NEG = -1e30


K1_BLK = 1024
K1_STEPS = 20


def _k1_body(fs_ref, feat_ref, fsnew_ref, alpha_ref):
    i = pl.program_id(0)
    x = fs_ref[...]
    f = feat_ref[...]
    rows = i * K1_BLK + jax.lax.broadcasted_iota(jnp.int32, (K1_BLK, 1), 0)
    fsnew_ref[...] = jnp.where(rows == NROWS, f[None, :], x)
    a = jnp.dot(x, f, preferred_element_type=jnp.float32)
    valid = (i * K1_BLK + jax.lax.iota(jnp.int32, K1_BLK)) < NROWS
    alpha_ref[...] = jnp.where(valid, a, _NEG).reshape(K1_BLK // 128, 128)


def _run_k1(fs2, feature):
    return pl.pallas_call(
        _k1_body,
        grid=(K1_STEPS,),
        in_specs=[
            pl.BlockSpec((K1_BLK, D_IN), lambda i: (i, 0)),
            pl.BlockSpec((D_IN,), lambda i: (0,)),
        ],
        out_specs=[
            pl.BlockSpec((K1_BLK, D_IN), lambda i: (i, 0)),
            pl.BlockSpec((K1_BLK // 128, 128), lambda i: (i, 0)),
        ],
        out_shape=[
            jax.ShapeDtypeStruct((NROWS + 1, D_IN), jnp.float32),
            jax.ShapeDtypeStruct((K1_STEPS * K1_BLK // 128, 128), jnp.float32),
        ],
    )(fs2, feature)


NPAD = K1_STEPS * K1_BLK


def _k2_body(t_ref, alpha_ref, ts_ref, w_ref):
    a = alpha_ref[...]
    t = t_ref[0]
    hi0 = jnp.max(a) + 1.0
    lo0 = jnp.min(jnp.where(a > 0.9 * _NEG, a, -hi0)) - 1.0

    def step(_, carry):
        lo, hi = carry
        mid = 0.5 * (lo + hi)
        cnt = jnp.sum(jnp.where(a >= mid, 1.0, 0.0))
        big = cnt >= float(TOPK)
        return (jnp.where(big, mid, lo), jnp.where(big, hi, mid))

    lo, hi = jax.lax.fori_loop(0, 64, step, (lo0, hi0))
    thr = lo
    sel = a >= thr
    ts_d = t - ts_ref[...]
    s = a * jnp.exp(jnp.float32(math.log(DECAY)) * ts_d)
    m = jnp.max(jnp.where(sel, s, _NEG))
    e = jnp.where(sel, jnp.exp(s - m), 0.0)
    w_ref[...] = e / jnp.sum(e)


def _run_k2(t_arr, alpha2, ts2):
    return pl.pallas_call(
        _k2_body,
        in_specs=[
            pl.BlockSpec(memory_space=pltpu.SMEM),
            pl.BlockSpec((NPAD // 128, 128), lambda: (0, 0)),
            pl.BlockSpec((NPAD // 128, 128), lambda: (0, 0)),
        ],
        out_specs=pl.BlockSpec((NPAD // 128, 128), lambda: (0, 0)),
        out_shape=jax.ShapeDtypeStruct((NPAD // 128, 128), jnp.float32),
    )(t_arr, alpha2, ts2)


K3_BLK = 512
K3_STEPS = 40


def _k3_body(hs_ref, w_ref, feat_ref, beta_ref, x_ref,
             W_ih_ref, W_hh_ref, b_ih_ref, b_hh_ref,
             Wo_f_ref, Wo_a_ref, Wo_h_ref, wo_len_ref, b_o_ref,
             hsnew_ref, out_ref, acc_ref):
    i = pl.program_id(0)
    hsb = hs_ref[...]
    wb = w_ref[...].reshape(1, K3_BLK)

    @pl.when(i == 0)
    def _init():
        acc_ref[...] = jnp.zeros_like(acc_ref)

    @pl.when(i < K3_STEPS - 1)
    def _acc():
        acc_ref[...] += jnp.dot(wb, hsb, preferred_element_type=jnp.float32)

    @pl.when(i == K3_STEPS - 1)
    def _acc_last():
        rows2 = (K3_STEPS - 1) * K3_BLK + jax.lax.broadcasted_iota(
            jnp.int32, (K3_BLK, 1), 0)
        hsb_m = jnp.where(rows2 < NROWS, hsb, 0.0)
        acc_ref[...] += jnp.dot(wb, hsb_m, preferred_element_type=jnp.float32)

    @pl.when(i < K3_STEPS - 1)
    def _copy():
        hsnew_ref[...] = hsb

    @pl.when(i == K3_STEPS - 1)
    def _epilogue():
        h0 = hsb[NROWS - 1 - (K3_STEPS - 1) * K3_BLK]
        x = x_ref[...]
        gi = jnp.dot(W_ih_ref[...], x, preferred_element_type=jnp.float32)
        gh = jnp.dot(W_hh_ref[...], h0, preferred_element_type=jnp.float32)
        gi = gi + b_ih_ref[...]
        gh = gh + b_hh_ref[...]
        r = jax.nn.sigmoid(gi[0:HID] + gh[0:HID])
        z = jax.nn.sigmoid(gi[HID:2 * HID] + gh[HID:2 * HID])
        nn_ = jnp.tanh(gi[2 * HID:3 * HID] + r * gh[2 * HID:3 * HID])
        h_new = (1.0 - z) * nn_ + z * h0
        rows = (K3_STEPS - 1) * K3_BLK + jax.lax.broadcasted_iota(
            jnp.int32, (K3_BLK, 1), 0)
        hsnew_ref[...] = jnp.where(rows == NROWS, h_new[None, :], hsb)
        attn = acc_ref[...]
        feat = feat_ref[...]
        pred = (jnp.dot(Wo_f_ref[...], feat, preferred_element_type=jnp.float32)
                + jnp.dot(Wo_a_ref[...], attn[0],
                          preferred_element_type=jnp.float32)
                + jnp.dot(Wo_h_ref[...], h0, preferred_element_type=jnp.float32)
                + wo_len_ref[...] * float(TOPK)
                + b_o_ref[...])
        v = pred * beta_ref[...]
        mx = jnp.max(v)
        lse = jnp.log(jnp.sum(jnp.exp(v - mx))) + mx
        out_ref[...] = (v - lse).reshape(1, D_OUT)


def _full(shape):
    return pl.BlockSpec(shape, lambda i: tuple(0 for _ in shape))


def _run_k3(hs2, w_flat, feature, beta, x, W_ih, W_hh, b_ih, b_hh,
            Wo_f, Wo_a, Wo_h, wo_len, b_o):
    return pl.pallas_call(
        _k3_body,
        grid=(K3_STEPS,),
        in_specs=[
            pl.BlockSpec((K3_BLK, HID), lambda i: (i, 0)),
            pl.BlockSpec((K3_BLK,), lambda i: (i,)),
            _full((D_IN,)), _full((D_OUT,)), _full((D_IN + D_OUT + 1,)),
            _full((3 * HID, D_IN + D_OUT + 1)), _full((3 * HID, HID)),
            _full((3 * HID,)), _full((3 * HID,)),
            _full((D_OUT, D_IN)), _full((D_OUT, HID)), _full((D_OUT, HID)),
            _full((D_OUT,)), _full((D_OUT,)),
        ],
        out_specs=[
            pl.BlockSpec((K3_BLK, HID), lambda i: (i, 0)),
            pl.BlockSpec((1, D_OUT), lambda i: (0, 0)),
        ],
        out_shape=[
            jax.ShapeDtypeStruct((NROWS + 1, HID), jnp.float32),
            jax.ShapeDtypeStruct((1, D_OUT), jnp.float32),
        ],
        scratch_shapes=[pltpu.VMEM((1, HID), jnp.float32)],
    )(hs2, w_flat, feature, beta, x, W_ih, W_hh, b_ih, b_hh,
      Wo_f, Wo_a, Wo_h, wo_len, b_o)


def kernel(feature, beta, time, fs, hs, ts, father,
           W_ih, W_hh, b_ih, b_hh, W_o, b_o):
    t = jnp.float32(time)
    fs2 = fs.reshape(NROWS, D_IN)
    hs2 = hs.reshape(NROWS, HID)

    fs_new2, alpha2 = _run_k1(fs2, feature)

    ts_pad = jnp.pad(ts, (0, NPAD - NROWS)).reshape(NPAD // 128, 128)
    w2 = _run_k2(t.reshape(1), alpha2, ts_pad)
    w_flat = w2.reshape(NPAD)

    x = jnp.concatenate([feature, beta, father])
    Wo_f = W_o[:, :D_IN]
    Wo_a = W_o[:, D_IN:D_IN + HID]
    Wo_h = W_o[:, D_IN + HID:D_IN + 2 * HID]
    wo_len = W_o[:, D_IN + 2 * HID]
    hs_new2, out = _run_k3(hs2, w_flat, feature, beta, x,
                           W_ih, W_hh, b_ih, b_hh,
                           Wo_f, Wo_a, Wo_h, wo_len, b_o)

    fs_new = fs_new2.reshape((NROWS + 1) * D_IN)
    hs_new = hs_new2.reshape(NROWS + 1, 1, HID)
    ts_new = jnp.concatenate([ts, t.reshape(1)])
    return (out, fs_new, hs_new, ts_new)


# speedup vs baseline: 1.3886x; 1.2252x over previous
import math

import jax
import jax.numpy as jnp
from jax.experimental import pallas as pl
from jax.experimental.pallas import tpu as pltpu

D_IN = 512
HID = 1024
D_OUT = 512
TOPK = 1024
NROWS = 20000
DECAY = 0.99

_NEG = -1e30


K1_BLK = 1024
K1_FLAT = K1_BLK * D_IN
K1_STEPS = 20
_TAIL_OFF = NROWS * D_IN - (K1_STEPS - 1) * K1_FLAT


def _k1_body(fs_ref, feat_ref, fsnew_ref, alpha_ref):
    i = pl.program_id(0)
    x = fs_ref[...]
    fsnew_ref[...] = x
    f = feat_ref[...]

    @pl.when(i == K1_STEPS - 1)
    def _tail():
        fsnew_ref[pl.ds(_TAIL_OFF, D_IN)] = f

    y = x.reshape(K1_BLK, D_IN)
    a = jnp.dot(y, f, preferred_element_type=jnp.float32)
    valid = (i * K1_BLK + jax.lax.iota(jnp.int32, K1_BLK)) < NROWS
    alpha_ref[...] = jnp.where(valid, a, _NEG).reshape(K1_BLK // 128, 128)


def _run_k1(fs_flat, feature):
    return pl.pallas_call(
        _k1_body,
        grid=(K1_STEPS,),
        in_specs=[
            pl.BlockSpec((K1_FLAT,), lambda i: (i,)),
            pl.BlockSpec((D_IN,), lambda i: (0,)),
        ],
        out_specs=[
            pl.BlockSpec((K1_FLAT,), lambda i: (i,)),
            pl.BlockSpec((K1_BLK // 128, 128), lambda i: (i, 0)),
        ],
        out_shape=[
            jax.ShapeDtypeStruct(((NROWS + 1) * D_IN,), jnp.float32),
            jax.ShapeDtypeStruct((K1_STEPS * K1_BLK // 128, 128), jnp.float32),
        ],
    )(fs_flat, feature)


NPAD = K1_STEPS * K1_BLK


def _k2_body(t_ref, alpha_ref, ts_ref, w_ref):
    a = alpha_ref[...]
    t = t_ref[0]
    hi0 = jnp.max(a) + 1.0
    lo0 = jnp.min(jnp.where(a > 0.9 * _NEG, a, -hi0)) - 1.0

    def step(_, carry):
        lo, hi = carry
        mid = 0.5 * (lo + hi)
        cnt = jnp.sum(jnp.where(a >= mid, 1.0, 0.0))
        big = cnt >= float(TOPK)
        return (jnp.where(big, mid, lo), jnp.where(big, hi, mid))

    lo, hi = jax.lax.fori_loop(0, 64, step, (lo0, hi0))
    thr = lo
    sel = a >= thr
    ts_d = t - ts_ref[...]
    s = a * jnp.exp(jnp.float32(math.log(DECAY)) * ts_d)
    m = jnp.max(jnp.where(sel, s, _NEG))
    e = jnp.where(sel, jnp.exp(s - m), 0.0)
    w_ref[...] = e / jnp.sum(e)


def _run_k2(t_arr, alpha2, ts2):
    return pl.pallas_call(
        _k2_body,
        in_specs=[
            pl.BlockSpec(memory_space=pltpu.SMEM),
            pl.BlockSpec((NPAD // 128, 128), lambda: (0, 0)),
            pl.BlockSpec((NPAD // 128, 128), lambda: (0, 0)),
        ],
        out_specs=pl.BlockSpec((NPAD // 128, 128), lambda: (0, 0)),
        out_shape=jax.ShapeDtypeStruct((NPAD // 128, 128), jnp.float32),
    )(t_arr, alpha2, ts2)


K3_BLK = 512
K3_STEPS = 40


def _k3_body(hs_ref, w_ref, feat_ref, beta_ref, x_ref,
             W_ih_ref, W_hh_ref, b_ih_ref, b_hh_ref,
             Wo_f_ref, Wo_a_ref, Wo_h_ref, wo_len_ref, b_o_ref,
             hsnew_ref, out_ref, acc_ref):
    i = pl.program_id(0)
    hsb = hs_ref[...]
    wb = w_ref[...].reshape(1, K3_BLK)

    @pl.when(i == 0)
    def _init():
        acc_ref[...] = jnp.zeros_like(acc_ref)

    @pl.when(i < K3_STEPS - 1)
    def _acc():
        acc_ref[...] += jnp.dot(wb, hsb, preferred_element_type=jnp.float32)

    @pl.when(i == K3_STEPS - 1)
    def _acc_last():
        rows2 = (K3_STEPS - 1) * K3_BLK + jax.lax.broadcasted_iota(
            jnp.int32, (K3_BLK, 1), 0)
        hsb_m = jnp.where(rows2 < NROWS, hsb, 0.0)
        acc_ref[...] += jnp.dot(wb, hsb_m, preferred_element_type=jnp.float32)

    @pl.when(i < K3_STEPS - 1)
    def _copy():
        hsnew_ref[...] = hsb

    @pl.when(i == K3_STEPS - 1)
    def _epilogue():
        h0 = hsb[NROWS - 1 - (K3_STEPS - 1) * K3_BLK]
        x = x_ref[...]
        gi = jnp.dot(W_ih_ref[...], x, preferred_element_type=jnp.float32)
        gh = jnp.dot(W_hh_ref[...], h0, preferred_element_type=jnp.float32)
        gi = gi + b_ih_ref[...]
        gh = gh + b_hh_ref[...]
        r = jax.nn.sigmoid(gi[0:HID] + gh[0:HID])
        z = jax.nn.sigmoid(gi[HID:2 * HID] + gh[HID:2 * HID])
        nn_ = jnp.tanh(gi[2 * HID:3 * HID] + r * gh[2 * HID:3 * HID])
        h_new = (1.0 - z) * nn_ + z * h0
        rows = (K3_STEPS - 1) * K3_BLK + jax.lax.broadcasted_iota(
            jnp.int32, (K3_BLK, 1), 0)
        hsnew_ref[...] = jnp.where(rows == NROWS, h_new[None, :], hsb)
        attn = acc_ref[...]
        feat = feat_ref[...]
        pred = (jnp.dot(Wo_f_ref[...], feat, preferred_element_type=jnp.float32)
                + jnp.dot(Wo_a_ref[...], attn[0],
                          preferred_element_type=jnp.float32)
                + jnp.dot(Wo_h_ref[...], h0, preferred_element_type=jnp.float32)
                + wo_len_ref[...] * float(TOPK)
                + b_o_ref[...])
        v = pred * beta_ref[...]
        mx = jnp.max(v)
        lse = jnp.log(jnp.sum(jnp.exp(v - mx))) + mx
        out_ref[...] = (v - lse).reshape(1, D_OUT)


def _full(shape):
    return pl.BlockSpec(shape, lambda i: tuple(0 for _ in shape))


def _run_k3(hs2, w_flat, feature, beta, x, W_ih, W_hh, b_ih, b_hh,
            Wo_f, Wo_a, Wo_h, wo_len, b_o):
    return pl.pallas_call(
        _k3_body,
        grid=(K3_STEPS,),
        in_specs=[
            pl.BlockSpec((K3_BLK, HID), lambda i: (i, 0)),
            pl.BlockSpec((K3_BLK,), lambda i: (i,)),
            _full((D_IN,)), _full((D_OUT,)), _full((D_IN + D_OUT + 1,)),
            _full((3 * HID, D_IN + D_OUT + 1)), _full((3 * HID, HID)),
            _full((3 * HID,)), _full((3 * HID,)),
            _full((D_OUT, D_IN)), _full((D_OUT, HID)), _full((D_OUT, HID)),
            _full((D_OUT,)), _full((D_OUT,)),
        ],
        out_specs=[
            pl.BlockSpec((K3_BLK, HID), lambda i: (i, 0)),
            pl.BlockSpec((1, D_OUT), lambda i: (0, 0)),
        ],
        out_shape=[
            jax.ShapeDtypeStruct((NROWS + 1, HID), jnp.float32),
            jax.ShapeDtypeStruct((1, D_OUT), jnp.float32),
        ],
        scratch_shapes=[pltpu.VMEM((1, HID), jnp.float32)],
    )(hs2, w_flat, feature, beta, x, W_ih, W_hh, b_ih, b_hh,
      Wo_f, Wo_a, Wo_h, wo_len, b_o)


def kernel(feature, beta, time, fs, hs, ts, father,
           W_ih, W_hh, b_ih, b_hh, W_o, b_o):
    t = jnp.float32(time)
    hs2 = hs.reshape(NROWS, HID)

    fs_new, alpha2 = _run_k1(fs, feature)

    ts_pad = jnp.pad(ts, (0, NPAD - NROWS)).reshape(NPAD // 128, 128)
    w2 = _run_k2(t.reshape(1), alpha2, ts_pad)
    w_flat = w2.reshape(NPAD)

    x = jnp.concatenate([feature, beta, father])
    Wo_f = W_o[:, :D_IN]
    Wo_a = W_o[:, D_IN:D_IN + HID]
    Wo_h = W_o[:, D_IN + HID:D_IN + 2 * HID]
    wo_len = W_o[:, D_IN + 2 * HID]
    hs_new2, out = _run_k3(hs2, w_flat, feature, beta, x,
                           W_ih, W_hh, b_ih, b_hh,
                           Wo_f, Wo_a, Wo_h, wo_len, b_o)

    hs_new = hs_new2.reshape(NROWS + 1, 1, HID)
    ts_new = jnp.concatenate([ts, t.reshape(1)])
    return (out, fs_new, hs_new, ts_new)


# speedup vs baseline: 2.2754x; 1.6386x over previous
import math

import jax
import jax.numpy as jnp
from jax.experimental import pallas as pl
from jax.experimental.pallas import tpu as pltpu

D_IN = 512
HID = 1024
D_OUT = 512
TOPK = 1024
NROWS = 20000
DECAY = 0.99

_NEG = -1e30


K1_BLK = 1024
K1_FLAT = K1_BLK * D_IN
K1_STEPS = 20
NPAD = K1_STEPS * K1_BLK
_TAIL_OFF = NROWS * D_IN - (K1_STEPS - 1) * K1_FLAT


def _k1_body(fs_ref, feat_ref, ts_ref, fsnew_ref, alpha_ref, tsp_ref):
    i = pl.program_id(0)
    x = fs_ref[...]
    fsnew_ref[...] = x
    f = feat_ref[...]

    @pl.when(i == K1_STEPS - 1)
    def _tail():
        fsnew_ref[pl.ds(_TAIL_OFF, D_IN)] = f

    y = x.reshape(K1_BLK, D_IN)
    a = jnp.dot(y, f, preferred_element_type=jnp.float32)
    valid = (i * K1_BLK + jax.lax.iota(jnp.int32, K1_BLK)) < NROWS
    alpha_ref[...] = jnp.where(valid, a, _NEG)
    tsp_ref[...] = jnp.where(valid, ts_ref[...], 0.0)


def _run_k1(fs_flat, feature, ts):
    return pl.pallas_call(
        _k1_body,
        grid=(K1_STEPS,),
        in_specs=[
            pl.BlockSpec((K1_FLAT,), lambda i: (i,)),
            pl.BlockSpec((D_IN,), lambda i: (0,)),
            pl.BlockSpec((K1_BLK,), lambda i: (i,)),
        ],
        out_specs=[
            pl.BlockSpec((K1_FLAT,), lambda i: (i,)),
            pl.BlockSpec((K1_BLK,), lambda i: (i,)),
            pl.BlockSpec((K1_BLK,), lambda i: (i,)),
        ],
        out_shape=[
            jax.ShapeDtypeStruct(((NROWS + 1) * D_IN,), jnp.float32),
            jax.ShapeDtypeStruct((NPAD,), jnp.float32),
            jax.ShapeDtypeStruct((NPAD,), jnp.float32),
        ],
    )(fs_flat, feature, ts)


def _k2_body(t_ref, alpha_ref, ts_ref, w_ref):
    a = alpha_ref[...]
    t = t_ref[0]
    hi0 = jnp.max(a) + 1.0
    lo0 = jnp.min(jnp.where(a > 0.9 * _NEG, a, -hi0)) - 1.0

    def step(_, carry):
        lo, hi = carry
        mid = 0.5 * (lo + hi)
        cnt = jnp.sum(jnp.where(a >= mid, 1.0, 0.0))
        big = cnt >= float(TOPK)
        return (jnp.where(big, mid, lo), jnp.where(big, hi, mid))

    lo, hi = jax.lax.fori_loop(0, 64, step, (lo0, hi0))
    thr = lo
    sel = a >= thr
    ts_d = t - ts_ref[...]
    s = a * jnp.exp(jnp.float32(math.log(DECAY)) * ts_d)
    m = jnp.max(jnp.where(sel, s, _NEG))
    e = jnp.where(sel, jnp.exp(s - m), 0.0)
    w_ref[...] = e / jnp.sum(e)


def _run_k2(t_arr, alpha1, ts1):
    return pl.pallas_call(
        _k2_body,
        in_specs=[
            pl.BlockSpec(memory_space=pltpu.SMEM),
            pl.BlockSpec((NPAD,), lambda: (0,)),
            pl.BlockSpec((NPAD,), lambda: (0,)),
        ],
        out_specs=pl.BlockSpec((NPAD,), lambda: (0,)),
        out_shape=jax.ShapeDtypeStruct((NPAD,), jnp.float32),
    )(t_arr, alpha1, ts1)


K3_BLK = 512
K3_STEPS = 40


def _k3_body(hs_ref, w_ref, feat_ref, beta_ref, x_ref,
             W_ih_ref, W_hh_ref, b_ih_ref, b_hh_ref,
             Wo_f_ref, Wo_a_ref, Wo_h_ref, wo_len_ref, b_o_ref,
             hsnew_ref, out_ref, acc_ref):
    i = pl.program_id(0)
    hsb = hs_ref[...]
    wb = w_ref[...].reshape(1, K3_BLK)

    @pl.when(i == 0)
    def _init():
        acc_ref[...] = jnp.zeros_like(acc_ref)

    @pl.when(i < K3_STEPS - 1)
    def _acc():
        acc_ref[...] += jnp.dot(wb, hsb, preferred_element_type=jnp.float32)

    @pl.when(i < K3_STEPS - 1)
    def _copy():
        hsnew_ref[...] = hsb

    @pl.when(i == K3_STEPS - 1)
    def _epilogue():
        rows = (K3_STEPS - 1) * K3_BLK + jax.lax.broadcasted_iota(
            jnp.int32, (K3_BLK, 1), 0)
        hsb_m = jnp.where(rows < NROWS, hsb, 0.0)
        acc_ref[...] += jnp.dot(wb, hsb_m, preferred_element_type=jnp.float32)

        h0 = hsb[NROWS - 1 - (K3_STEPS - 1) * K3_BLK]
        x = x_ref[...]
        gi = jnp.dot(W_ih_ref[...], x, preferred_element_type=jnp.float32)
        gh = jnp.dot(W_hh_ref[...], h0, preferred_element_type=jnp.float32)
        gi = gi + b_ih_ref[...]
        gh = gh + b_hh_ref[...]
        r = jax.nn.sigmoid(gi[0:HID] + gh[0:HID])
        z = jax.nn.sigmoid(gi[HID:2 * HID] + gh[HID:2 * HID])
        nn_ = jnp.tanh(gi[2 * HID:3 * HID] + r * gh[2 * HID:3 * HID])
        h_new = (1.0 - z) * nn_ + z * h0
        hsnew_ref[...] = jnp.where(rows == NROWS, h_new[None, :], hsb)
        attn = acc_ref[...]
        feat = feat_ref[...]
        pred = (jnp.dot(Wo_f_ref[...], feat, preferred_element_type=jnp.float32)
                + jnp.dot(Wo_a_ref[...], attn[0],
                          preferred_element_type=jnp.float32)
                + jnp.dot(Wo_h_ref[...], h0, preferred_element_type=jnp.float32)
                + wo_len_ref[...] * float(TOPK)
                + b_o_ref[...])
        v = pred * beta_ref[...]
        mx = jnp.max(v)
        lse = jnp.log(jnp.sum(jnp.exp(v - mx))) + mx
        out_ref[...] = (v - lse).reshape(1, D_OUT)


def _full(shape):
    return pl.BlockSpec(shape, lambda i: tuple(0 for _ in shape))


def _run_k3(hs3, w1, feature, beta, x, W_ih, W_hh, b_ih, b_hh,
            Wo_f, Wo_a, Wo_h, wo_len, b_o):
    return pl.pallas_call(
        _k3_body,
        grid=(K3_STEPS,),
        in_specs=[
            pl.BlockSpec((K3_BLK, None, HID), lambda i: (i, 0, 0)),
            pl.BlockSpec((K3_BLK,), lambda i: (i,)),
            _full((D_IN,)), _full((D_OUT,)), _full((D_IN + D_OUT + 1,)),
            _full((3 * HID, D_IN + D_OUT + 1)), _full((3 * HID, HID)),
            _full((3 * HID,)), _full((3 * HID,)),
            _full((D_OUT, D_IN)), _full((D_OUT, HID)), _full((D_OUT, HID)),
            _full((D_OUT,)), _full((D_OUT,)),
        ],
        out_specs=[
            pl.BlockSpec((K3_BLK, None, HID), lambda i: (i, 0, 0)),
            pl.BlockSpec((1, D_OUT), lambda i: (0, 0)),
        ],
        out_shape=[
            jax.ShapeDtypeStruct((NROWS + 1, 1, HID), jnp.float32),
            jax.ShapeDtypeStruct((1, D_OUT), jnp.float32),
        ],
        scratch_shapes=[pltpu.VMEM((1, HID), jnp.float32)],
    )(hs3, w1, feature, beta, x, W_ih, W_hh, b_ih, b_hh,
      Wo_f, Wo_a, Wo_h, wo_len, b_o)


def kernel(feature, beta, time, fs, hs, ts, father,
           W_ih, W_hh, b_ih, b_hh, W_o, b_o):
    t = jnp.float32(time)

    fs_new, alpha1, ts1 = _run_k1(fs, feature, ts)
    w1 = _run_k2(t.reshape(1), alpha1, ts1)

    x = jnp.concatenate([feature, beta, father])
    Wo_f = W_o[:, :D_IN]
    Wo_a = W_o[:, D_IN:D_IN + HID]
    Wo_h = W_o[:, D_IN + HID:D_IN + 2 * HID]
    wo_len = W_o[:, D_IN + 2 * HID]
    hs_new, out = _run_k3(hs, w1, feature, beta, x,
                          W_ih, W_hh, b_ih, b_hh,
                          Wo_f, Wo_a, Wo_h, wo_len, b_o)

    ts_new = jnp.concatenate([ts, t.reshape(1)])
    return (out, fs_new, hs_new, ts_new)


# speedup vs baseline: 2.2976x; 1.0098x over previous
import math

import jax
import jax.numpy as jnp
from jax.experimental import pallas as pl
from jax.experimental.pallas import tpu as pltpu

D_IN = 512
HID = 1024
D_OUT = 512
TOPK = 1024
NROWS = 20000
DECAY = 0.99

_NEG = -1e30


K1_BLK = 1024
K1_FLAT = K1_BLK * D_IN
K1_STEPS = 20
NPAD = K1_STEPS * K1_BLK
_TAIL_OFF = NROWS * D_IN - (K1_STEPS - 1) * K1_FLAT


def _k1_body(fs_ref, feat_ref, ts_ref, fsnew_ref, alpha_ref, tsp_ref):
    i = pl.program_id(0)
    x = fs_ref[...]
    fsnew_ref[...] = x
    f = feat_ref[...]

    @pl.when(i == K1_STEPS - 1)
    def _tail():
        fsnew_ref[pl.ds(_TAIL_OFF, D_IN)] = f

    y = x.reshape(K1_BLK, D_IN)
    a = jnp.dot(y, f, preferred_element_type=jnp.float32)
    valid = (i * K1_BLK + jax.lax.iota(jnp.int32, K1_BLK)) < NROWS
    alpha_ref[...] = jnp.where(valid, a, _NEG)
    tsp_ref[...] = jnp.where(valid, ts_ref[...], 0.0)


def _run_k1(fs_flat, feature, ts):
    return pl.pallas_call(
        _k1_body,
        grid=(K1_STEPS,),
        in_specs=[
            pl.BlockSpec((K1_FLAT,), lambda i: (i,)),
            pl.BlockSpec((D_IN,), lambda i: (0,)),
            pl.BlockSpec((K1_BLK,), lambda i: (i,)),
        ],
        out_specs=[
            pl.BlockSpec((K1_FLAT,), lambda i: (i,)),
            pl.BlockSpec((K1_BLK,), lambda i: (i,)),
            pl.BlockSpec((K1_BLK,), lambda i: (i,)),
        ],
        out_shape=[
            jax.ShapeDtypeStruct(((NROWS + 1) * D_IN,), jnp.float32),
            jax.ShapeDtypeStruct((NPAD,), jnp.float32),
            jax.ShapeDtypeStruct((NPAD,), jnp.float32),
        ],
    )(fs_flat, feature, ts)


K3_BLK = 512
K3_STEPS = 40


def _k3_body(t_ref, hs_ref, alpha_ref, ts_ref, feat_ref, beta_ref, x_ref,
             W_ih_ref, W_hh_ref, b_ih_ref, b_hh_ref,
             Wo_f_ref, Wo_a_ref, Wo_h_ref, wo_len_ref, b_o_ref,
             hsnew_ref, out_ref, acc_ref, w_ref):
    i = pl.program_id(0)
    hsb = hs_ref[...]

    @pl.when(i == 0)
    def _init():
        acc_ref[...] = jnp.zeros_like(acc_ref)
        a = alpha_ref[...]
        t = t_ref[0]
        hi0 = jnp.max(a) + 1.0
        lo0 = jnp.min(jnp.where(a > 0.9 * _NEG, a, -hi0)) - 1.0

        def step(_, carry):
            lo, hi = carry
            mid = 0.5 * (lo + hi)
            cnt = jnp.sum(jnp.where(a >= mid, 1.0, 0.0))
            big = cnt >= float(TOPK)
            return (jnp.where(big, mid, lo), jnp.where(big, hi, mid))

        lo, _hi = jax.lax.fori_loop(0, 64, step, (lo0, hi0))
        sel = a >= lo
        ts_d = t - ts_ref[...]
        s = a * jnp.exp(jnp.float32(math.log(DECAY)) * ts_d)
        m = jnp.max(jnp.where(sel, s, _NEG))
        e = jnp.where(sel, jnp.exp(s - m), 0.0)
        w_ref[...] = e / jnp.sum(e)

    wb = w_ref[pl.ds(i * K3_BLK, K3_BLK)].reshape(1, K3_BLK)

    @pl.when(i < K3_STEPS - 1)
    def _acc():
        acc_ref[...] += jnp.dot(wb, hsb, preferred_element_type=jnp.float32)

    @pl.when(i < K3_STEPS - 1)
    def _copy():
        hsnew_ref[...] = hsb

    @pl.when(i == K3_STEPS - 1)
    def _epilogue():
        rows = (K3_STEPS - 1) * K3_BLK + jax.lax.broadcasted_iota(
            jnp.int32, (K3_BLK, 1), 0)
        hsb_m = jnp.where(rows < NROWS, hsb, 0.0)
        acc_ref[...] += jnp.dot(wb, hsb_m, preferred_element_type=jnp.float32)

        h0 = hsb[NROWS - 1 - (K3_STEPS - 1) * K3_BLK]
        x = x_ref[...]
        gi = jnp.dot(W_ih_ref[...], x, preferred_element_type=jnp.float32)
        gh = jnp.dot(W_hh_ref[...], h0, preferred_element_type=jnp.float32)
        gi = gi + b_ih_ref[...]
        gh = gh + b_hh_ref[...]
        r = jax.nn.sigmoid(gi[0:HID] + gh[0:HID])
        z = jax.nn.sigmoid(gi[HID:2 * HID] + gh[HID:2 * HID])
        nn_ = jnp.tanh(gi[2 * HID:3 * HID] + r * gh[2 * HID:3 * HID])
        h_new = (1.0 - z) * nn_ + z * h0
        hsnew_ref[...] = jnp.where(rows == NROWS, h_new[None, :], hsb)
        attn = acc_ref[...]
        feat = feat_ref[...]
        pred = (jnp.dot(Wo_f_ref[...], feat, preferred_element_type=jnp.float32)
                + jnp.dot(Wo_a_ref[...], attn[0],
                          preferred_element_type=jnp.float32)
                + jnp.dot(Wo_h_ref[...], h0, preferred_element_type=jnp.float32)
                + wo_len_ref[...] * float(TOPK)
                + b_o_ref[...])
        v = pred * beta_ref[...]
        mx = jnp.max(v)
        lse = jnp.log(jnp.sum(jnp.exp(v - mx))) + mx
        out_ref[...] = (v - lse).reshape(1, D_OUT)


def _full(shape):
    return pl.BlockSpec(shape, lambda i: tuple(0 for _ in shape))


def _run_k3(t_arr, hs3, alpha1, ts1, feature, beta, x, W_ih, W_hh, b_ih, b_hh,
            Wo_f, Wo_a, Wo_h, wo_len, b_o):
    return pl.pallas_call(
        _k3_body,
        grid=(K3_STEPS,),
        in_specs=[
            pl.BlockSpec(memory_space=pltpu.SMEM),
            pl.BlockSpec((K3_BLK, None, HID), lambda i: (i, 0, 0)),
            _full((NPAD,)), _full((NPAD,)),
            _full((D_IN,)), _full((D_OUT,)), _full((D_IN + D_OUT + 1,)),
            _full((3 * HID, D_IN + D_OUT + 1)), _full((3 * HID, HID)),
            _full((3 * HID,)), _full((3 * HID,)),
            _full((D_OUT, D_IN)), _full((D_OUT, HID)), _full((D_OUT, HID)),
            _full((D_OUT,)), _full((D_OUT,)),
        ],
        out_specs=[
            pl.BlockSpec((K3_BLK, None, HID), lambda i: (i, 0, 0)),
            pl.BlockSpec((1, D_OUT), lambda i: (0, 0)),
        ],
        out_shape=[
            jax.ShapeDtypeStruct((NROWS + 1, 1, HID), jnp.float32),
            jax.ShapeDtypeStruct((1, D_OUT), jnp.float32),
        ],
        scratch_shapes=[pltpu.VMEM((1, HID), jnp.float32),
                        pltpu.VMEM((NPAD,), jnp.float32)],
    )(t_arr, hs3, alpha1, ts1, feature, beta, x, W_ih, W_hh, b_ih, b_hh,
      Wo_f, Wo_a, Wo_h, wo_len, b_o)


def kernel(feature, beta, time, fs, hs, ts, father,
           W_ih, W_hh, b_ih, b_hh, W_o, b_o):
    t = jnp.float32(time)

    fs_new, alpha1, ts1 = _run_k1(fs, feature, ts)

    x = jnp.concatenate([feature, beta, father])
    Wo_f = W_o[:, :D_IN]
    Wo_a = W_o[:, D_IN:D_IN + HID]
    Wo_h = W_o[:, D_IN + HID:D_IN + 2 * HID]
    wo_len = W_o[:, D_IN + 2 * HID]
    hs_new, out = _run_k3(t.reshape(1), hs, alpha1, ts1, feature, beta, x,
                          W_ih, W_hh, b_ih, b_hh,
                          Wo_f, Wo_a, Wo_h, wo_len, b_o)

    ts_new = jnp.concatenate([ts, t.reshape(1)])
    return (out, fs_new, hs_new, ts_new)


# speedup vs baseline: 2.5559x; 1.1124x over previous
import math

import jax
import jax.numpy as jnp
from jax.experimental import pallas as pl
from jax.experimental.pallas import tpu as pltpu

D_IN = 512
HID = 1024
D_OUT = 512
TOPK = 1024
NROWS = 20000
DECAY = 0.99

_NEG = -1e30


K1_BLK = 2048
K1_FLAT = K1_BLK * D_IN
K1_STEPS = 10
NPAD = K1_STEPS * K1_BLK
_TAIL_OFF = NROWS * D_IN - (K1_STEPS - 1) * K1_FLAT


def _k1_body(fs_ref, feat_ref, ts_ref, fsnew_ref, alpha_ref, tsp_ref):
    i = pl.program_id(0)
    x = fs_ref[...]
    fsnew_ref[...] = x
    f = feat_ref[...]

    @pl.when(i == K1_STEPS - 1)
    def _tail():
        fsnew_ref[pl.ds(_TAIL_OFF, D_IN)] = f

    y = x.reshape(K1_BLK, D_IN)
    a = jnp.dot(y, f, preferred_element_type=jnp.float32)
    valid = (i * K1_BLK + jax.lax.iota(jnp.int32, K1_BLK)) < NROWS
    alpha_ref[...] = jnp.where(valid, a, _NEG)
    tsp_ref[...] = jnp.where(valid, ts_ref[...], 0.0)


def _run_k1(fs_flat, feature, ts):
    return pl.pallas_call(
        _k1_body,
        grid=(K1_STEPS,),
        in_specs=[
            pl.BlockSpec((K1_FLAT,), lambda i: (i,)),
            pl.BlockSpec((D_IN,), lambda i: (0,)),
            pl.BlockSpec((K1_BLK,), lambda i: (i,)),
        ],
        out_specs=[
            pl.BlockSpec((K1_FLAT,), lambda i: (i,)),
            pl.BlockSpec((K1_BLK,), lambda i: (i,)),
            pl.BlockSpec((K1_BLK,), lambda i: (i,)),
        ],
        out_shape=[
            jax.ShapeDtypeStruct(((NROWS + 1) * D_IN,), jnp.float32),
            jax.ShapeDtypeStruct((NPAD,), jnp.float32),
            jax.ShapeDtypeStruct((NPAD,), jnp.float32),
        ],
    )(fs_flat, feature, ts)


K3_BLK = 1024
K3_STEPS = 20


def _k3_body(t_ref, hs_ref, alpha_ref, ts_ref, feat_ref, beta_ref, x_ref,
             W_ih_ref, W_hh_ref, b_ih_ref, b_hh_ref,
             Wo_f_ref, Wo_a_ref, Wo_h_ref, wo_len_ref, b_o_ref,
             hsnew_ref, out_ref, acc_ref, w_ref):
    i = pl.program_id(0)
    hsb = hs_ref[...]

    @pl.when(i == 0)
    def _init():
        acc_ref[...] = jnp.zeros_like(acc_ref)
        a = alpha_ref[...]
        t = t_ref[0]
        hi0 = jnp.max(a) + 1.0
        lo0 = jnp.min(jnp.where(a > 0.9 * _NEG, a, -hi0)) - 1.0

        def step(_, carry):
            lo, hi = carry
            mid = 0.5 * (lo + hi)
            cnt = jnp.sum(jnp.where(a >= mid, 1.0, 0.0))
            big = cnt >= float(TOPK)
            return (jnp.where(big, mid, lo), jnp.where(big, hi, mid))

        lo, _hi = jax.lax.fori_loop(0, 64, step, (lo0, hi0))
        sel = a >= lo
        ts_d = t - ts_ref[...]
        s = a * jnp.exp(jnp.float32(math.log(DECAY)) * ts_d)
        m = jnp.max(jnp.where(sel, s, _NEG))
        e = jnp.where(sel, jnp.exp(s - m), 0.0)
        w_ref[...] = e / jnp.sum(e)

    wb = w_ref[pl.ds(i * K3_BLK, K3_BLK)].reshape(1, K3_BLK)

    @pl.when(i < K3_STEPS - 1)
    def _acc():
        acc_ref[...] += jnp.dot(wb, hsb, preferred_element_type=jnp.float32)

    @pl.when(i < K3_STEPS - 1)
    def _copy():
        hsnew_ref[...] = hsb

    @pl.when(i == K3_STEPS - 1)
    def _epilogue():
        rows = (K3_STEPS - 1) * K3_BLK + jax.lax.broadcasted_iota(
            jnp.int32, (K3_BLK, 1), 0)
        hsb_m = jnp.where(rows < NROWS, hsb, 0.0)
        acc_ref[...] += jnp.dot(wb, hsb_m, preferred_element_type=jnp.float32)

        h0 = hsb[NROWS - 1 - (K3_STEPS - 1) * K3_BLK]
        x = x_ref[...]
        gi = jnp.dot(W_ih_ref[...], x, preferred_element_type=jnp.float32)
        gh = jnp.dot(W_hh_ref[...], h0, preferred_element_type=jnp.float32)
        gi = gi + b_ih_ref[...]
        gh = gh + b_hh_ref[...]
        r = jax.nn.sigmoid(gi[0:HID] + gh[0:HID])
        z = jax.nn.sigmoid(gi[HID:2 * HID] + gh[HID:2 * HID])
        nn_ = jnp.tanh(gi[2 * HID:3 * HID] + r * gh[2 * HID:3 * HID])
        h_new = (1.0 - z) * nn_ + z * h0
        hsnew_ref[...] = jnp.where(rows == NROWS, h_new[None, :], hsb)
        attn = acc_ref[...]
        feat = feat_ref[...]
        pred = (jnp.dot(Wo_f_ref[...], feat, preferred_element_type=jnp.float32)
                + jnp.dot(Wo_a_ref[...], attn[0],
                          preferred_element_type=jnp.float32)
                + jnp.dot(Wo_h_ref[...], h0, preferred_element_type=jnp.float32)
                + wo_len_ref[...] * float(TOPK)
                + b_o_ref[...])
        v = pred * beta_ref[...]
        mx = jnp.max(v)
        lse = jnp.log(jnp.sum(jnp.exp(v - mx))) + mx
        out_ref[...] = (v - lse).reshape(1, D_OUT)


def _full(shape):
    return pl.BlockSpec(shape, lambda i: tuple(0 for _ in shape))


def _run_k3(t_arr, hs3, alpha1, ts1, feature, beta, x, W_ih, W_hh, b_ih, b_hh,
            Wo_f, Wo_a, Wo_h, wo_len, b_o):
    return pl.pallas_call(
        _k3_body,
        grid=(K3_STEPS,),
        in_specs=[
            pl.BlockSpec(memory_space=pltpu.SMEM),
            pl.BlockSpec((K3_BLK, None, HID), lambda i: (i, 0, 0)),
            _full((NPAD,)), _full((NPAD,)),
            _full((D_IN,)), _full((D_OUT,)), _full((D_IN + D_OUT + 1,)),
            _full((3 * HID, D_IN + D_OUT + 1)), _full((3 * HID, HID)),
            _full((3 * HID,)), _full((3 * HID,)),
            _full((D_OUT, D_IN)), _full((D_OUT, HID)), _full((D_OUT, HID)),
            _full((D_OUT,)), _full((D_OUT,)),
        ],
        out_specs=[
            pl.BlockSpec((K3_BLK, None, HID), lambda i: (i, 0, 0)),
            pl.BlockSpec((1, D_OUT), lambda i: (0, 0)),
        ],
        out_shape=[
            jax.ShapeDtypeStruct((NROWS + 1, 1, HID), jnp.float32),
            jax.ShapeDtypeStruct((1, D_OUT), jnp.float32),
        ],
        scratch_shapes=[pltpu.VMEM((1, HID), jnp.float32),
                        pltpu.VMEM((NPAD,), jnp.float32)],
    )(t_arr, hs3, alpha1, ts1, feature, beta, x, W_ih, W_hh, b_ih, b_hh,
      Wo_f, Wo_a, Wo_h, wo_len, b_o)


def kernel(feature, beta, time, fs, hs, ts, father,
           W_ih, W_hh, b_ih, b_hh, W_o, b_o):
    t = jnp.float32(time)

    fs_new, alpha1, ts1 = _run_k1(fs, feature, ts)

    x = jnp.concatenate([feature, beta, father])
    Wo_f = W_o[:, :D_IN]
    Wo_a = W_o[:, D_IN:D_IN + HID]
    Wo_h = W_o[:, D_IN + HID:D_IN + 2 * HID]
    wo_len = W_o[:, D_IN + 2 * HID]
    hs_new, out = _run_k3(t.reshape(1), hs, alpha1, ts1, feature, beta, x,
                          W_ih, W_hh, b_ih, b_hh,
                          Wo_f, Wo_a, Wo_h, wo_len, b_o)

    ts_new = jnp.concatenate([ts, t.reshape(1)])
    return (out, fs_new, hs_new, ts_new)
